# Initial kernel scaffold; baseline (speedup 1.0000x reference)
import jax
import jax.numpy as jnp
from jax.experimental import pallas as pl


def kernel(feature_values, feature_idx, seq_lens, weights_first_order):
    raise NotImplementedError("write your pallas kernel here")


# speedup vs baseline: 66.5135x; 66.5135x over previous
import functools

import jax
import jax.numpy as jnp
from jax import lax
from jax.experimental import pallas as pl
from jax.experimental.pallas import tpu as pltpu
from jax.experimental.pallas import tpu_sc as plsc

FEATURE_SIZE = 1000000
FIELD_SIZE = 26
BATCH = 4096
MAX_LEN = 20

NUM_WORKERS = 32
BPW = BATCH // NUM_WORKERS
CHUNK = BPW * MAX_LEN
CHUNK_ROWS = CHUNK // 128
OUT_PER_W = BPW * FIELD_SIZE


def _sc_kernel(vals_hbm, idx_hbm, seq_hbm, table_hbm, out_hbm,
               idx_buf, v_buf, w_buf, seq_buf, out_buf, sem):
    info = plsc.get_sparse_core_info()
    nc = info.num_cores
    wid = lax.axis_index("s") * nc + lax.axis_index("c")
    b0 = wid * BPW

    lane = lax.iota(jnp.int32, 16)
    lane20 = lane * MAX_LEN
    lane26 = lane * FIELD_SIZE

    pltpu.sync_copy(seq_hbm.at[pl.ds(b0 * FIELD_SIZE, OUT_PER_W)], seq_buf)

    def field_body(f, carry):
        flat0 = (f * NUM_WORKERS + wid) * CHUNK
        pltpu.sync_copy(idx_hbm.at[pl.ds(flat0, CHUNK)], idx_buf)
        pltpu.sync_copy(vals_hbm.at[pl.ds(flat0, CHUNK)], v_buf)
        pltpu.async_copy(table_hbm.at[idx_buf], w_buf, sem).wait()

        def group_body(g, c):
            acc = jnp.zeros((16,), jnp.float32)
            base = g * (16 * MAX_LEN)
            for l in range(MAX_LEN):
                flat = base + l + lane20
                w = plsc.load_gather(w_buf, [flat])
                v = plsc.load_gather(v_buf, [flat])
                acc = acc + w * v
            i_out = (g * 16) * FIELD_SIZE + lane26 + f
            s = plsc.load_gather(seq_buf, [i_out]).astype(jnp.float32)
            plsc.store_scatter(out_buf, [i_out], acc / s)
            return c

        lax.fori_loop(0, BPW // 16, group_body, 0)
        return carry

    lax.fori_loop(0, FIELD_SIZE, field_body, 0)

    pltpu.sync_copy(out_buf, out_hbm.at[pl.ds(b0 * FIELD_SIZE, OUT_PER_W)])


@jax.jit
def _first_order(vals2d, idx2d, seq_flat, table_flat):
    mesh = plsc.VectorSubcoreMesh(core_axis_name="c", subcore_axis_name="s")
    run = functools.partial(
        pl.kernel,
        out_type=jax.ShapeDtypeStruct((BATCH * FIELD_SIZE,), jnp.float32),
        mesh=mesh,
        compiler_params=pltpu.CompilerParams(needs_layout_passes=False),
        scratch_types=[
            pltpu.VMEM((CHUNK,), jnp.int32),
            pltpu.VMEM((CHUNK,), jnp.float32),
            pltpu.VMEM((CHUNK,), jnp.float32),
            pltpu.VMEM((OUT_PER_W,), jnp.int32),
            pltpu.VMEM((OUT_PER_W,), jnp.float32),
            pltpu.SemaphoreType.DMA,
        ],
    )(_sc_kernel)
    return run(vals2d, idx2d, seq_flat, table_flat)


def kernel(feature_values, feature_idx, seq_lens, weights_first_order):
    n = FIELD_SIZE * BATCH * MAX_LEN
    vals_flat = feature_values.reshape(n)
    idx_flat = feature_idx.astype(jnp.int32).reshape(n)
    seq_flat = seq_lens.reshape(BATCH * FIELD_SIZE)
    table_flat = weights_first_order.reshape(FEATURE_SIZE + 2)
    out = _first_order(vals_flat, idx_flat, seq_flat, table_flat)
    return out.reshape(BATCH, FIELD_SIZE)
